# baseline (speedup 1.0000x reference)
import jax
import jax.numpy as jnp
from jax import lax
from jax.experimental import pallas as pl
from jax.experimental.pallas import tpu as pltpu
from jax.experimental.pallas import tpu_sc as plsc

VOCAB = 100000
HIDDEN = 768
BATCH = 64
SEQ = 512
LN_EPS = 1e-5

L = 16
NW = 32
POS_PER_W = SEQ // NW
NJ = HIDDEN // L


def _rsqrt(v):
    i = lax.bitcast_convert_type(v, jnp.int32)
    y = lax.bitcast_convert_type(jnp.int32(0x5F3759DF) - (i >> 1), jnp.float32)
    for _ in range(3):
        y = y * (1.5 - 0.5 * v * y * y)
    return y


def _lanesum(v):
    iota = lax.iota(jnp.int32, L)
    dnums = lax.GatherDimensionNumbers(
        offset_dims=(), collapsed_slice_dims=(0,), start_index_map=(0,))
    for k in (8, 4, 2, 1):
        perm = (iota ^ k).reshape(L, 1)
        v = v + lax.gather(v, perm, dnums, (1,),
                           mode=lax.GatherScatterMode.PROMISE_IN_BOUNDS)
    return v


NBUF = 2


def _body(x_hbm, we_hbm, pe_hbm, g_hbm, b_hbm, out_hbm,
          idx_v, pos_v, gam_v, bet_v, inb, outb, gsem, osem):
    c = lax.axis_index("c")
    s = lax.axis_index("s")
    wid = s * 2 + c
    s0 = wid * POS_PER_W

    pltpu.sync_copy(x_hbm, idx_v)
    pltpu.sync_copy(pe_hbm.at[pl.ds(s0, POS_PER_W)], pos_v)
    pltpu.sync_copy(g_hbm, gam_v)
    pltpu.sync_copy(b_hbm, bet_v)

    def gather(b, slot):
        return pltpu.make_async_copy(
            we_hbm.at[idx_v.at[b, pl.ds(s0, POS_PER_W)]], inb[slot], gsem[slot])

    def outcopy(b, slot, buf=None):
        src = outb[slot] if buf is None else buf
        return pltpu.make_async_copy(
            src, out_hbm.at[b, pl.ds(s0, POS_PER_W)], osem[slot])

    NACC = 4

    def ln_rows(src, dst):
        def pass1(t):
            accs = [jnp.zeros((L,), jnp.float32)] * NACC
            accqs = [jnp.zeros((L,), jnp.float32)] * NACC
            for j in range(NJ):
                v = src[t, pl.ds(j * L, L)] + pos_v[t, pl.ds(j * L, L)]
                dst[t, pl.ds(j * L, L)] = v
                k = j % NACC
                accs[k] = accs[k] + v
                accqs[k] = accqs[k] + v * v
            acc = (accs[0] + accs[1]) + (accs[2] + accs[3])
            accq = (accqs[0] + accqs[1]) + (accqs[2] + accqs[3])
            mean_v = _lanesum(acc) * (1.0 / HIDDEN)
            var_v = _lanesum(accq) * (1.0 / HIDDEN) - mean_v * mean_v
            rstd = _rsqrt(var_v + LN_EPS)
            return mean_v, rstd

        def pass2(t, mean_v, rstd):
            for j in range(NJ):
                v = dst[t, pl.ds(j * L, L)]
                g = gam_v[pl.ds(j * L, L)]
                be = bet_v[pl.ds(j * L, L)]
                dst[t, pl.ds(j * L, L)] = (v - mean_v) * rstd * g + be

        def pair_body(tt, _):
            t0 = tt * 2
            t1 = t0 + 1
            m0, r0 = pass1(t0)
            m1, r1 = pass1(t1)
            pass2(t0, m0, r0)
            pass2(t1, m1, r1)
            return 0

        lax.fori_loop(0, POS_PER_W // 2, pair_body, 0)

    for slot in range(NBUF):
        gather(slot, slot).start()

    def b_group(q, _):
        for slot in range(NBUF):
            b = q * NBUF + slot

            @pl.when(q > 0)
            def _wait_out():
                outcopy(b, slot).wait()

            gather(b, slot).wait()
            ln_rows(inb[slot], outb[slot])

            @pl.when(b + NBUF < BATCH)
            def _refill():
                gather(b + NBUF, slot).start()

            outcopy(b, slot).start()
        return 0

    lax.fori_loop(0, BATCH // NBUF, b_group, 0)
    for slot in range(NBUF):
        outcopy(BATCH - NBUF + slot, slot).wait()


@jax.jit
def kernel(x, word_emb, pos_emb, ln_gamma, ln_beta):
    mesh = plsc.VectorSubcoreMesh(core_axis_name="c", subcore_axis_name="s")
    run = pl.kernel(
        _body,
        out_type=jax.ShapeDtypeStruct((BATCH, SEQ, HIDDEN), jnp.float32),
        mesh=mesh,
        scratch_types=[
            pltpu.VMEM((BATCH, SEQ), jnp.int32),
            pltpu.VMEM((POS_PER_W, HIDDEN), jnp.float32),
            pltpu.VMEM((HIDDEN,), jnp.float32),
            pltpu.VMEM((HIDDEN,), jnp.float32),
            [pltpu.VMEM((POS_PER_W, HIDDEN), jnp.float32)
             for _ in range(NBUF)],
            [pltpu.VMEM((POS_PER_W, HIDDEN), jnp.float32)
             for _ in range(NBUF)],
            [pltpu.SemaphoreType.DMA for _ in range(NBUF)],
            [pltpu.SemaphoreType.DMA for _ in range(NBUF)],
        ],
    )
    return run(x, word_emb, pos_emb, ln_gamma, ln_beta)


# speedup vs baseline: 2.2320x; 2.2320x over previous
import jax
import jax.numpy as jnp
from jax import lax
from jax.experimental import pallas as pl
from jax.experimental.pallas import tpu as pltpu
from jax.experimental.pallas import tpu_sc as plsc

VOCAB = 100000
HIDDEN = 768
BATCH = 64
SEQ = 512
LN_EPS = 1e-5

L = 16
NW = 32
POS_PER_W = SEQ // NW
NSLOT = 4


def _sc_gather_body(x_hbm, we_hbm, out_hbm, idx_v, bufs, gsems, osems):
    c = lax.axis_index("c")
    s = lax.axis_index("s")
    wid = s * 2 + c
    s0 = wid * POS_PER_W

    pltpu.sync_copy(x_hbm, idx_v)

    def gather(b, slot):
        return pltpu.make_async_copy(
            we_hbm.at[idx_v.at[b, pl.ds(s0, POS_PER_W)]], bufs[slot],
            gsems[slot])

    def outcopy(b, slot):
        return pltpu.make_async_copy(
            bufs[slot], out_hbm.at[b, pl.ds(s0, POS_PER_W)], osems[slot])

    for slot in range(NSLOT):
        gather(slot, slot).start()

    def b_group(q, _):
        for slot in range(NSLOT):
            b = q * NSLOT + slot
            gather(b, slot).wait()
            outcopy(b, slot).start()
        for slot in range(NSLOT):
            b = q * NSLOT + slot
            outcopy(b, slot).wait()

            @pl.when(b + NSLOT < BATCH)
            def _refill():
                gather(b + NSLOT, slot).start()
        return 0

    lax.fori_loop(0, BATCH // NSLOT, b_group, 0)


def _sc_gather(x, word_emb):
    mesh = plsc.VectorSubcoreMesh(core_axis_name="c", subcore_axis_name="s")
    run = pl.kernel(
        _sc_gather_body,
        out_type=jax.ShapeDtypeStruct((BATCH, SEQ, HIDDEN), jnp.float32),
        mesh=mesh,
        scratch_types=[
            pltpu.VMEM((BATCH, SEQ), jnp.int32),
            [pltpu.VMEM((POS_PER_W, HIDDEN), jnp.float32)
             for _ in range(NSLOT)],
            [pltpu.SemaphoreType.DMA for _ in range(NSLOT)],
            [pltpu.SemaphoreType.DMA for _ in range(NSLOT)],
        ],
    )
    return run(x, word_emb)


def _tc_ln_body(g_ref, pos_ref, gam_ref, bet_ref, out_ref):
    v = g_ref[0] + pos_ref[...]
    mean = jnp.mean(v, axis=-1, keepdims=True)
    cent = v - mean
    var = jnp.mean(cent * cent, axis=-1, keepdims=True)
    normed = cent * lax.rsqrt(var + LN_EPS)
    out_ref[0] = normed * gam_ref[0] + bet_ref[0]


def _tc_ln(g, pos_emb, ln_gamma, ln_beta):
    return pl.pallas_call(
        _tc_ln_body,
        grid=(BATCH,),
        in_specs=[
            pl.BlockSpec((1, SEQ, HIDDEN), lambda i: (i, 0, 0)),
            pl.BlockSpec((SEQ, HIDDEN), lambda i: (0, 0)),
            pl.BlockSpec((1, HIDDEN), lambda i: (0, 0)),
            pl.BlockSpec((1, HIDDEN), lambda i: (0, 0)),
        ],
        out_specs=pl.BlockSpec((1, SEQ, HIDDEN), lambda i: (i, 0, 0)),
        out_shape=jax.ShapeDtypeStruct((BATCH, SEQ, HIDDEN), jnp.float32),
    )(g, pos_emb, ln_gamma.reshape(1, HIDDEN), ln_beta.reshape(1, HIDDEN))


@jax.jit
def kernel(x, word_emb, pos_emb, ln_gamma, ln_beta):
    g = _sc_gather(x, word_emb)
    return _tc_ln(g, pos_emb, ln_gamma, ln_beta)


# speedup vs baseline: 2.2871x; 1.0247x over previous
import jax
import jax.numpy as jnp
from jax import lax
from jax.experimental import pallas as pl
from jax.experimental.pallas import tpu as pltpu
from jax.experimental.pallas import tpu_sc as plsc

VOCAB = 100000
HIDDEN = 768
BATCH = 64
SEQ = 512
LN_EPS = 1e-5

L = 16
NW = 32
POS_PER_W = SEQ // NW
NSLOT = 4


def _make_sc_gather_body(nb):
    def _sc_gather_body(x_hbm, we_hbm, out_hbm, idx_v, bufs, gsems, osems):
        c = lax.axis_index("c")
        s = lax.axis_index("s")
        wid = s * 2 + c
        s0 = wid * POS_PER_W

        pltpu.sync_copy(x_hbm, idx_v)

        def gather(b, slot):
            return pltpu.make_async_copy(
                we_hbm.at[idx_v.at[b, pl.ds(s0, POS_PER_W)]], bufs[slot],
                gsems[slot])

        def outcopy(b, slot):
            return pltpu.make_async_copy(
                bufs[slot], out_hbm.at[b, pl.ds(s0, POS_PER_W)], osems[slot])

        for slot in range(NSLOT):
            gather(slot, slot).start()

        def b_group(q, _):
            for slot in range(NSLOT):
                b = q * NSLOT + slot
                gather(b, slot).wait()
                outcopy(b, slot).start()
            for slot in range(NSLOT):
                b = q * NSLOT + slot
                outcopy(b, slot).wait()

                @pl.when(b + NSLOT < nb)
                def _refill():
                    gather(b + NSLOT, slot).start()
            return 0

        lax.fori_loop(0, nb // NSLOT, b_group, 0)

    return _sc_gather_body


def _sc_gather(x, word_emb):
    nb = x.shape[0]
    mesh = plsc.VectorSubcoreMesh(core_axis_name="c", subcore_axis_name="s")
    run = pl.kernel(
        _make_sc_gather_body(nb),
        out_type=jax.ShapeDtypeStruct((nb, SEQ, HIDDEN), jnp.float32),
        mesh=mesh,
        scratch_types=[
            pltpu.VMEM((nb, SEQ), jnp.int32),
            [pltpu.VMEM((POS_PER_W, HIDDEN), jnp.float32)
             for _ in range(NSLOT)],
            [pltpu.SemaphoreType.DMA for _ in range(NSLOT)],
            [pltpu.SemaphoreType.DMA for _ in range(NSLOT)],
        ],
    )
    return run(x, word_emb)


def _tc_ln_compute(g_ref, pos_ref, gam_ref, bet_ref, out_ref):
    v = g_ref[0] + pos_ref[...]
    mean = jnp.mean(v, axis=-1, keepdims=True)
    cent = v - mean
    var = jnp.mean(cent * cent, axis=-1, keepdims=True)
    normed = cent * lax.rsqrt(var + LN_EPS)
    out_ref[0] = normed * gam_ref[0] + bet_ref[0]


def _tc_ln_body_first(g_ref, pos_ref, gam_ref, bet_ref, out_ref):
    _tc_ln_compute(g_ref, pos_ref, gam_ref, bet_ref, out_ref)


def _tc_ln_body_chain(prev_ref, g_ref, pos_ref, gam_ref, bet_ref, out_ref):
    del prev_ref
    _tc_ln_compute(g_ref, pos_ref, gam_ref, bet_ref, out_ref)


def _tc_ln_chunk(prev, g, pos_emb, gamma2d, beta2d, b0):
    nb = g.shape[0]
    small_specs = [
        pl.BlockSpec((SEQ, HIDDEN), lambda i: (0, 0)),
        pl.BlockSpec((1, HIDDEN), lambda i: (0, 0)),
        pl.BlockSpec((1, HIDDEN), lambda i: (0, 0)),
    ]
    out_spec = pl.BlockSpec((1, SEQ, HIDDEN), lambda i: (b0 + i, 0, 0))
    out_shape = jax.ShapeDtypeStruct((BATCH, SEQ, HIDDEN), jnp.float32)
    if prev is None:
        return pl.pallas_call(
            _tc_ln_body_first,
            grid=(nb,),
            in_specs=[pl.BlockSpec((1, SEQ, HIDDEN), lambda i: (i, 0, 0))]
            + small_specs,
            out_specs=out_spec,
            out_shape=out_shape,
        )(g, pos_emb, gamma2d, beta2d)
    return pl.pallas_call(
        _tc_ln_body_chain,
        grid=(nb,),
        in_specs=[pl.BlockSpec(memory_space=pl.ANY),
                  pl.BlockSpec((1, SEQ, HIDDEN), lambda i: (i, 0, 0))]
        + small_specs,
        out_specs=out_spec,
        out_shape=out_shape,
        input_output_aliases={0: 0},
    )(prev, g, pos_emb, gamma2d, beta2d)


NCHUNK = 4


@jax.jit
def kernel(x, word_emb, pos_emb, ln_gamma, ln_beta):
    cb = BATCH // NCHUNK
    gamma2d = ln_gamma.reshape(1, HIDDEN)
    beta2d = ln_beta.reshape(1, HIDDEN)
    gs = [_sc_gather(x[c * cb:(c + 1) * cb], word_emb)
          for c in range(NCHUNK)]
    out = None
    for c in range(NCHUNK):
        out = _tc_ln_chunk(out, gs[c], pos_emb, gamma2d, beta2d, c * cb)
    return out
